# Initial kernel scaffold; baseline (speedup 1.0000x reference)
import jax
import jax.numpy as jnp
from jax.experimental import pallas as pl


def kernel(ind_2, p1, p3, d3, basis, W_pi, b_pi, W_ii, W_pp, W_pp1, b_pp1, W_px):
    raise NotImplementedError("write your pallas kernel here")


# speedup vs baseline: 8.6257x; 8.6257x over previous
import functools

import jax
import jax.numpy as jnp
from jax import lax
from jax.experimental import pallas as pl
from jax.experimental.pallas import tpu as pltpu
from jax.experimental.pallas import tpu_sc as plsc

N, E, C, NB = 10000, 160000, 128, 4
D3 = 3 * C

NC, NS = 2, 16
NW = NC * NS

_MESH = plsc.VectorSubcoreMesh(core_axis_name="c", subcore_axis_name="s")

CH = 128
EPW = E // NW
GCH = EPW // CH
GT = EPW - GCH * CH

EPT = E // NS
SCH = EPT // CH
ST = EPT - SCH * CH
RPT = N // NS


def _wid():
    return lax.axis_index("s") * NC + lax.axis_index("c")


@functools.partial(
    pl.kernel,
    out_type=[
        jax.ShapeDtypeStruct((E, C), jnp.float32),
        jax.ShapeDtypeStruct((E, C), jnp.float32),
    ],
    mesh=_MESH,
    scratch_types=[
        pltpu.VMEM((CH,), jnp.int32),
        pltpu.VMEM((CH,), jnp.int32),
        pltpu.VMEM((GT,), jnp.int32),
        pltpu.VMEM((GT,), jnp.int32),
        pltpu.VMEM((CH, C), jnp.float32),
        pltpu.VMEM((CH, C), jnp.float32),
        pltpu.VMEM((GT, C), jnp.float32),
        pltpu.VMEM((GT, C), jnp.float32),
        pltpu.SemaphoreType.DMA,
        pltpu.SemaphoreType.DMA,
    ],
)
def _gather_p1(p1_hbm, ii_hbm, ij_hbm, gi_hbm, gj_hbm,
               idx_i, idx_j, idx_it, idx_jt, rows_i, rows_j, rows_it, rows_jt,
               sem_i, sem_j):
    base = _wid() * EPW

    def chunk(off, n, ivr, jvr, ribuf, rjbuf):
        pltpu.sync_copy(ii_hbm.at[pl.ds(off, n)], ivr)
        pltpu.sync_copy(ij_hbm.at[pl.ds(off, n)], jvr)
        ci = pltpu.async_copy(p1_hbm.at[ivr], ribuf, sem_i)
        cj = pltpu.async_copy(p1_hbm.at[jvr], rjbuf, sem_j)
        ci.wait()
        cj.wait()
        pltpu.sync_copy(ribuf, gi_hbm.at[pl.ds(off, n)])
        pltpu.sync_copy(rjbuf, gj_hbm.at[pl.ds(off, n)])

    def body(k, carry):
        chunk(base + k * CH, CH, idx_i, idx_j, rows_i, rows_j)
        return carry

    lax.fori_loop(0, GCH, body, 0)
    chunk(base + GCH * CH, GT, idx_it, idx_jt, rows_it, rows_jt)


BE = 1280
GE = E // BE


def _edge_dense_body(gi_ref, gj_ref, bas_ref, w_ref, b_ref, wii_ref, i1_ref):
    w = w_ref[...]
    acc = jnp.dot(gi_ref[...], w[0:C, :], preferred_element_type=jnp.float32)
    acc = acc + jnp.dot(gj_ref[...], w[C:2 * C, :],
                        preferred_element_type=jnp.float32)
    h = jnp.tanh(acc + b_ref[...])
    bas = bas_ref[...]
    pre = h[:, 0:C] * bas[:, 0:1]
    for nb in range(1, NB):
        pre = pre + h[:, nb * C:(nb + 1) * C] * bas[:, nb:nb + 1]
    i1_ref[...] = jnp.tanh(
        jnp.dot(pre, wii_ref[...], preferred_element_type=jnp.float32))


def _edge_dense(gi, gj, basis, w_pi_p, b_pi_p, w_ii):
    return pl.pallas_call(
        _edge_dense_body,
        grid=(GE,),
        in_specs=[
            pl.BlockSpec((BE, C), lambda g: (g, 0)),
            pl.BlockSpec((BE, C), lambda g: (g, 0)),
            pl.BlockSpec((BE, NB), lambda g: (g, 0)),
            pl.BlockSpec((2 * C, NB * C), lambda g: (0, 0)),
            pl.BlockSpec((1, NB * C), lambda g: (0, 0)),
            pl.BlockSpec((C, 2 * C), lambda g: (0, 0)),
        ],
        out_specs=pl.BlockSpec((BE, 2 * C), lambda g: (g, 0)),
        out_shape=jax.ShapeDtypeStruct((E, 2 * C), jnp.float32),
    )(gi, gj, basis, w_pi_p, b_pi_p, w_ii)


@functools.partial(
    pl.kernel,
    out_type=jax.ShapeDtypeStruct((E, D3), jnp.float32),
    mesh=_MESH,
    scratch_types=[
        pltpu.VMEM((CH,), jnp.int32),
        pltpu.VMEM((GT,), jnp.int32),
        pltpu.VMEM((CH, D3), jnp.float32),
        pltpu.VMEM((GT, D3), jnp.float32),
        pltpu.VMEM((CH, C), jnp.float32),
        pltpu.VMEM((3 * CH + 16,), jnp.float32),
        pltpu.SemaphoreType.DMA,
    ],
)
def _edge3(p3_hbm, ij_hbm, d3f_hbm, i1_hbm, i3_hbm,
           idx_j, idx_jt, rows, rows_t, i1b, d3v, sem):
    base = _wid() * EPW

    def chunk(off, n, jvr, rbuf):
        pltpu.sync_copy(ij_hbm.at[pl.ds(off, n)], jvr)
        cp = pltpu.async_copy(p3_hbm.at[jvr], rbuf, sem)
        pltpu.sync_copy(d3f_hbm.at[pl.ds(3 * off, 3 * n)],
                        d3v.at[pl.ds(0, 3 * n)])
        pltpu.sync_copy(i1_hbm.at[pl.ds(off, n), pl.ds(C, C)],
                        i1b.at[pl.ds(0, n)])
        cp.wait()

        def row(b, carry):
            dvec = d3v[pl.ds(3 * b, 16)]
            for x in range(3):
                dv = jnp.full((16,), dvec[x], jnp.float32)
                for g in range(C // 16):
                    iv = i1b[b, pl.ds(g * 16, 16)]
                    pv = rbuf[b, pl.ds(x * C + g * 16, 16)]
                    rbuf[b, pl.ds(x * C + g * 16, 16)] = (pv + dv) * iv
            return carry

        lax.fori_loop(0, n, row, 0)
        pltpu.sync_copy(rbuf, i3_hbm.at[pl.ds(off, n)])

    def body(k, carry):
        chunk(base + k * CH, CH, idx_j, rows)
        return carry

    lax.fori_loop(0, GCH, body, 0)
    chunk(base + GCH * CH, GT, idx_jt, rows_t)


@functools.partial(
    pl.kernel,
    out_type=[
        jax.ShapeDtypeStruct((N, 2 * C), jnp.float32),
        jax.ShapeDtypeStruct((N, D3), jnp.float32),
    ],
    mesh=_MESH,
    scratch_types=[
        pltpu.VMEM((CH,), jnp.int32),
        pltpu.VMEM((ST,), jnp.int32),
        pltpu.VMEM((CH, C), jnp.float32),
        pltpu.VMEM((ST, C), jnp.float32),
        pltpu.VMEM((104, C), jnp.float32),
        pltpu.VMEM((104, C), jnp.float32),
        pltpu.VMEM_SHARED((N, C), jnp.float32),
    ],
)
def _scatter_all(i1_hbm, i3_hbm, idx_hbm, pn_hbm, p3a_hbm,
                 idxv, idxt, datav, datat, rbuf, zbuf, acc):
    cid = lax.axis_index("c")
    sid = lax.axis_index("s")
    zv = jnp.zeros((16,), jnp.float32)

    def zb(b, carry):
        for g in range(C // 16):
            zbuf[b, pl.ds(g * 16, 16)] = zv
        return carry

    lax.fori_loop(0, 104, zb, 0)

    ebase = sid * EPT
    WPT = 624

    def phase(data_hbm, dcol, out_hbm, ocol):
        for t in range(WPT // 104):
            pltpu.sync_copy(zbuf, acc.at[pl.ds(sid * WPT + t * 104, 104)])

        @pl.when(sid == 0)
        def _ztail():
            pltpu.sync_copy(zbuf.at[pl.ds(0, 16)], acc.at[pl.ds(NS * WPT, 16)])

        plsc.subcore_barrier()

        def chunk(off, n, ivr, dbuf):
            pltpu.sync_copy(idx_hbm.at[pl.ds(off, n)], ivr)
            pltpu.sync_copy(data_hbm.at[pl.ds(off, n), pl.ds(dcol, C)], dbuf)
            pltpu.sync_copy(dbuf, acc.at[ivr], add=True)

        def body(k, carry):
            chunk(ebase + k * CH, CH, idxv, datav)
            return carry

        lax.fori_loop(0, SCH, body, 0)
        chunk(ebase + SCH * CH, ST, idxt, datat)
        plsc.subcore_barrier()
        for t in range(WPT // 104):
            r0 = sid * WPT + t * 104
            pltpu.sync_copy(acc.at[pl.ds(r0, 104)], rbuf)
            pltpu.sync_copy(rbuf, out_hbm.at[pl.ds(r0, 104), pl.ds(ocol, C)])

        @pl.when(sid == 0)
        def _wtail():
            pltpu.sync_copy(acc.at[pl.ds(NS * WPT, 16)], rbuf.at[pl.ds(0, 16)])
            pltpu.sync_copy(rbuf.at[pl.ds(0, 16)],
                            out_hbm.at[pl.ds(NS * WPT, 16), pl.ds(ocol, C)])

        plsc.subcore_barrier()

    ccol = cid * C
    phase(i1_hbm, ccol, pn_hbm, ccol)
    phase(i3_hbm, ccol, p3a_hbm, ccol)

    @pl.when(cid == 0)
    def _x2_phase():
        phase(i3_hbm, 2 * C, p3a_hbm, 2 * C)


BN = 1000
GN = N // BN


def _node_body(pn_ref, p3a_ref, wpp_ref, wpx_ref, wpp1_ref, bpp1_ref,
               pa_ref, dot_ref, p3t_ref):
    wpx = wpx_ref[...]
    p3l0 = jnp.dot(p3a_ref[0], wpx, preferred_element_type=jnp.float32)
    p3l1 = jnp.dot(p3a_ref[1], wpx, preferred_element_type=jnp.float32)
    p3l2 = jnp.dot(p3a_ref[2], wpx, preferred_element_type=jnp.float32)
    dotted = p3l0 * p3l0 + p3l1 * p3l1 + p3l2 * p3l2
    p1n = jnp.tanh(
        jnp.dot(pn_ref[...], wpp_ref[...], preferred_element_type=jnp.float32))
    w1 = wpp1_ref[...]
    z = (jnp.dot(p1n, w1[0:C, :], preferred_element_type=jnp.float32)
         + jnp.dot(dotted, w1[C:2 * C, :], preferred_element_type=jnp.float32)
         + bpp1_ref[...])
    p1t1 = jnp.tanh(z)
    pa_ref[...] = p1t1[:, 0:C]
    dot_ref[...] = dotted
    pb = p1t1[:, C:2 * C]
    p3t_ref[0] = p3l0 * pb
    p3t_ref[1] = p3l1 * pb
    p3t_ref[2] = p3l2 * pb


def _node_dense(p1n_raw, p3a_x, w_pp, w_px, w_pp1, b_pp1):
    return pl.pallas_call(
        _node_body,
        grid=(GN,),
        in_specs=[
            pl.BlockSpec((BN, 2 * C), lambda g: (g, 0)),
            pl.BlockSpec((3, BN, C), lambda g: (0, g, 0)),
            pl.BlockSpec((2 * C, C), lambda g: (0, 0)),
            pl.BlockSpec((C, C), lambda g: (0, 0)),
            pl.BlockSpec((2 * C, 2 * C), lambda g: (0, 0)),
            pl.BlockSpec((1, 2 * C), lambda g: (0, 0)),
        ],
        out_specs=[
            pl.BlockSpec((BN, C), lambda g: (g, 0)),
            pl.BlockSpec((BN, C), lambda g: (g, 0)),
            pl.BlockSpec((3, BN, C), lambda g: (0, g, 0)),
        ],
        out_shape=[
            jax.ShapeDtypeStruct((N, C), jnp.float32),
            jax.ShapeDtypeStruct((N, C), jnp.float32),
            jax.ShapeDtypeStruct((3, N, C), jnp.float32),
        ],
    )(p1n_raw, p3a_x, w_pp, w_px, w_pp1, b_pp1)


def kernel(ind_2, p1, p3, d3, basis, W_pi, b_pi, W_ii, W_pp, W_pp1, b_pp1, W_px):
    ind_i = ind_2[:, 0]
    ind_j = ind_2[:, 1]
    p3f = p3.reshape(N, D3)

    w_pi_p = W_pi.reshape(2 * C, C, NB).transpose(0, 2, 1).reshape(2 * C, C * NB)
    b_pi_p = b_pi.reshape(C, NB).transpose(1, 0).reshape(1, C * NB)

    gi, gj = _gather_p1(p1, ind_i, ind_j)
    i1 = _edge_dense(gi, gj, basis, w_pi_p, b_pi_p, W_ii)
    i3f = _edge3(p3f, ind_j, d3.reshape(-1), i1)
    p1n_raw, p3af = _scatter_all(i1, i3f, ind_i)

    p3a_x = p3af.reshape(N, 3, C).transpose(1, 0, 2)
    pxt1_a, dotted, p3t_x = _node_dense(p1n_raw, p3a_x, W_pp, W_px, W_pp1,
                                        b_pp1.reshape(1, 2 * C))

    i3 = i3f.reshape(E, 3, C)
    p3t1 = p3t_x.transpose(1, 0, 2)
    return (pxt1_a, i1, i3, dotted, p3t1)


# speedup vs baseline: 11.5345x; 1.3372x over previous
import functools

import jax
import jax.numpy as jnp
from jax import lax
from jax.experimental import pallas as pl
from jax.experimental.pallas import tpu as pltpu
from jax.experimental.pallas import tpu_sc as plsc

N, E, C, NB = 10000, 160000, 128, 4
D3 = 3 * C

NC, NS = 2, 16
NW = NC * NS

_MESH = plsc.VectorSubcoreMesh(core_axis_name="c", subcore_axis_name="s")

CH = 128
EPW = E // NW
GCH = EPW // CH
GT = EPW - GCH * CH

EPT = E // NS
SCH = EPT // CH
ST = EPT - SCH * CH
RPT = N // NS


def _wid():
    return lax.axis_index("s") * NC + lax.axis_index("c")


@functools.partial(
    pl.kernel,
    out_type=[
        jax.ShapeDtypeStruct((E, C), jnp.float32),
        jax.ShapeDtypeStruct((E, C), jnp.float32),
    ],
    mesh=_MESH,
    scratch_types=[
        pltpu.VMEM((EPW,), jnp.int32),
        pltpu.VMEM((EPW,), jnp.int32),
        pltpu.VMEM((CH, C), jnp.float32),
        pltpu.VMEM((CH, C), jnp.float32),
        pltpu.VMEM((CH, C), jnp.float32),
        pltpu.VMEM((CH, C), jnp.float32),
        pltpu.VMEM((CH, C), jnp.float32),
        pltpu.VMEM((CH, C), jnp.float32),
        pltpu.VMEM((GT, C), jnp.float32),
        pltpu.VMEM((GT, C), jnp.float32),
        pltpu.SemaphoreType.DMA,
        pltpu.SemaphoreType.DMA,
        pltpu.SemaphoreType.DMA,
        pltpu.SemaphoreType.DMA,
        pltpu.SemaphoreType.DMA,
        pltpu.SemaphoreType.DMA,
        pltpu.SemaphoreType.DMA,
    ],
)
def _gather_p1(p1_hbm, ii_hbm, ij_hbm, gi_hbm, gj_hbm,
               iall, jall, ri0, ri1, ri2, rj0, rj1, rj2, rit, rjt,
               g0, g1, g2, o0, o1, o2, ts):
    base = _wid() * EPW
    pltpu.sync_copy(ii_hbm.at[pl.ds(base, EPW)], iall)
    pltpu.sync_copy(ij_hbm.at[pl.ds(base, EPW)], jall)
    ri = (ri0, ri1, ri2)
    rj = (rj0, rj1, rj2)
    gs = (g0, g1, g2)
    os = (o0, o1, o2)

    def issue(k, m):
        sl = pl.ds(k * CH, CH)
        pltpu.async_copy(p1_hbm.at[iall.at[sl]], ri[m], gs[m])
        pltpu.async_copy(p1_hbm.at[jall.at[sl]], rj[m], gs[m])

    def wait_g(m):
        pltpu.make_async_copy(p1_hbm.at[pl.ds(0, CH)], ri[m], gs[m]).wait()
        pltpu.make_async_copy(p1_hbm.at[pl.ds(0, CH)], rj[m], gs[m]).wait()

    def put(k, m):
        off = base + k * CH
        pltpu.async_copy(ri[m], gi_hbm.at[pl.ds(off, CH)], os[m])
        pltpu.async_copy(rj[m], gj_hbm.at[pl.ds(off, CH)], os[m])

    def drain_o(m):
        pltpu.make_async_copy(ri[m], gi_hbm.at[pl.ds(0, CH)], os[m]).wait()
        pltpu.make_async_copy(rj[m], gj_hbm.at[pl.ds(0, CH)], os[m]).wait()

    issue(0, 0)

    def body(i, carry):
        for b in range(3):
            k = 3 * i + b
            wait_g(b)
            put(k, b)
            nm = (b + 1) % 3
            if b == 2:
                @pl.when(i < (GCH // 3) - 1)
                def _issue_next():
                    drain_o(nm)
                    issue(k + 1, nm)
            else:
                @pl.when(i >= 1)
                def _drain_next():
                    drain_o(nm)

                issue(k + 1, nm)
        return carry

    lax.fori_loop(0, GCH // 3, body, 0)
    for m in range(3):
        drain_o(m)
    toff = base + GCH * CH
    ci = pltpu.async_copy(p1_hbm.at[iall.at[pl.ds(GCH * CH, GT)]], rit, ts)
    cj = pltpu.async_copy(p1_hbm.at[jall.at[pl.ds(GCH * CH, GT)]], rjt, ts)
    ci.wait()
    cj.wait()
    pltpu.sync_copy(rit, gi_hbm.at[pl.ds(toff, GT)])
    pltpu.sync_copy(rjt, gj_hbm.at[pl.ds(toff, GT)])


BE = 1280
GE = E // BE


def _edge_dense_body(gi_ref, gj_ref, bas_ref, w_ref, b_ref, wii_ref, i1_ref):
    w = w_ref[...]
    acc = jnp.dot(gi_ref[...], w[0:C, :], preferred_element_type=jnp.float32)
    acc = acc + jnp.dot(gj_ref[...], w[C:2 * C, :],
                        preferred_element_type=jnp.float32)
    h = jnp.tanh(acc + b_ref[...])
    bas = bas_ref[...]
    pre = h[:, 0:C] * bas[:, 0:1]
    for nb in range(1, NB):
        pre = pre + h[:, nb * C:(nb + 1) * C] * bas[:, nb:nb + 1]
    i1_ref[...] = jnp.tanh(
        jnp.dot(pre, wii_ref[...], preferred_element_type=jnp.float32))


def _edge_dense(gi, gj, basis, w_pi_p, b_pi_p, w_ii):
    return pl.pallas_call(
        _edge_dense_body,
        grid=(GE,),
        in_specs=[
            pl.BlockSpec((BE, C), lambda g: (g, 0)),
            pl.BlockSpec((BE, C), lambda g: (g, 0)),
            pl.BlockSpec((BE, NB), lambda g: (g, 0)),
            pl.BlockSpec((2 * C, NB * C), lambda g: (0, 0)),
            pl.BlockSpec((1, NB * C), lambda g: (0, 0)),
            pl.BlockSpec((C, 2 * C), lambda g: (0, 0)),
        ],
        out_specs=pl.BlockSpec((BE, 2 * C), lambda g: (g, 0)),
        out_shape=jax.ShapeDtypeStruct((E, 2 * C), jnp.float32),
    )(gi, gj, basis, w_pi_p, b_pi_p, w_ii)


C3 = 64
NC3 = EPW // C3
GT3 = EPW - NC3 * C3


@functools.partial(
    pl.kernel,
    out_type=jax.ShapeDtypeStruct((E, D3), jnp.float32),
    mesh=_MESH,
    scratch_types=[
        pltpu.VMEM((EPW,), jnp.int32),
        pltpu.VMEM((3 * EPW + 24,), jnp.float32),
        pltpu.VMEM((C3, D3), jnp.float32),
        pltpu.VMEM((C3, D3), jnp.float32),
        pltpu.VMEM((C3, D3), jnp.float32),
        pltpu.VMEM((C3, C), jnp.float32),
        pltpu.VMEM((C3, C), jnp.float32),
        pltpu.VMEM((C3, C), jnp.float32),
        pltpu.VMEM((GT3, D3), jnp.float32),
        pltpu.VMEM((GT3, C), jnp.float32),
        pltpu.SemaphoreType.DMA,
        pltpu.SemaphoreType.DMA,
        pltpu.SemaphoreType.DMA,
        pltpu.SemaphoreType.DMA,
        pltpu.SemaphoreType.DMA,
        pltpu.SemaphoreType.DMA,
        pltpu.SemaphoreType.DMA,
    ],
)
def _edge3(p3_hbm, ij_hbm, d3f_hbm, i1_hbm, i3_hbm,
           jall, d3a, r0, r1, r2, q0, q1, q2, rt, qt,
           g0, g1, g2, o0, o1, o2, ts):
    base = _wid() * EPW
    pltpu.sync_copy(ij_hbm.at[pl.ds(base, EPW)], jall)
    pltpu.sync_copy(d3f_hbm.at[pl.ds(3 * base, 3 * EPW)],
                    d3a.at[pl.ds(0, 3 * EPW)])
    rs = (r0, r1, r2)
    qs = (q0, q1, q2)
    gs = (g0, g1, g2)
    os = (o0, o1, o2)

    def issue(k, m):
        pltpu.async_copy(p3_hbm.at[jall.at[pl.ds(k * C3, C3)]], rs[m], gs[m])
        pltpu.async_copy(i1_hbm.at[pl.ds(base + k * C3, C3), pl.ds(C, C)],
                         qs[m], gs[m])

    def wait_g(m):
        pltpu.make_async_copy(p3_hbm.at[pl.ds(0, C3)], rs[m], gs[m]).wait()
        pltpu.make_async_copy(i1_hbm.at[pl.ds(0, C3), pl.ds(C, C)],
                              qs[m], gs[m]).wait()

    def drain_o(m):
        pltpu.make_async_copy(rs[m], i3_hbm.at[pl.ds(0, C3)], os[m]).wait()

    def compute(k, rbuf, qbuf, n):
        def row(b, carry):
            dvec = d3a[pl.ds(3 * (k * C3 + b), 16)]
            for x in range(3):
                dv = jnp.full((16,), dvec[x], jnp.float32)
                for g in range(C // 16):
                    iv = qbuf[b, pl.ds(g * 16, 16)]
                    pv = rbuf[b, pl.ds(x * C + g * 16, 16)]
                    rbuf[b, pl.ds(x * C + g * 16, 16)] = (pv + dv) * iv
            return carry

        lax.fori_loop(0, n, row, 0)

    issue(0, 0)

    def body(i, carry):
        for b in range(3):
            k = 3 * i + b
            wait_g(b)
            nm = (b + 1) % 3
            if b == 2:
                @pl.when(i < (NC3 // 3) - 1)
                def _issue_next():
                    drain_o(nm)
                    issue(k + 1, nm)
            else:
                @pl.when(i >= 1)
                def _drain_next():
                    drain_o(nm)

                issue(k + 1, nm)
            compute(k, rs[b], qs[b], C3)
            pltpu.async_copy(rs[b], i3_hbm.at[pl.ds(base + k * C3, C3)], os[b])
        return carry

    lax.fori_loop(0, NC3 // 3, body, 0)
    for m in range(3):
        drain_o(m)
    ci = pltpu.async_copy(p3_hbm.at[jall.at[pl.ds(NC3 * C3, GT3)]], rt, ts)
    cq = pltpu.async_copy(i1_hbm.at[pl.ds(base + NC3 * C3, GT3), pl.ds(C, C)],
                          qt, ts)
    ci.wait()
    cq.wait()
    compute(NC3, rt, qt, GT3)
    pltpu.sync_copy(rt, i3_hbm.at[pl.ds(base + NC3 * C3, GT3)])


@functools.partial(
    pl.kernel,
    out_type=[
        jax.ShapeDtypeStruct((N, 2 * C), jnp.float32),
        jax.ShapeDtypeStruct((N, D3), jnp.float32),
    ],
    mesh=_MESH,
    scratch_types=[
        pltpu.VMEM((CH,), jnp.int32),
        pltpu.VMEM((CH,), jnp.int32),
        pltpu.VMEM((ST,), jnp.int32),
        pltpu.VMEM((CH, C), jnp.float32),
        pltpu.VMEM((CH, C), jnp.float32),
        pltpu.VMEM((ST, C), jnp.float32),
        pltpu.VMEM((104, C), jnp.float32),
        pltpu.VMEM_SHARED((N, C), jnp.float32),
        pltpu.SemaphoreType.DMA,
        pltpu.SemaphoreType.DMA,
    ],
)
def _scatter_all(i1_hbm, i3_hbm, idx_hbm, pn_hbm, p3a_hbm,
                 idx0, idx1, idxt, dat0, dat1, datat, rbuf, acc, l0, l1):
    cid = lax.axis_index("c")
    sid = lax.axis_index("s")
    zv = jnp.zeros((16,), jnp.float32)

    ebase = sid * EPT
    WPT = 624

    def phase(data_hbm, dcol, out_hbm, ocol):
        def zb(b, carry):
            for g in range(C // 16):
                rbuf[b, pl.ds(g * 16, 16)] = zv
            return carry

        lax.fori_loop(0, 104, zb, 0)
        for t in range(WPT // 104):
            pltpu.sync_copy(rbuf, acc.at[pl.ds(sid * WPT + t * 104, 104)])

        @pl.when(sid == 0)
        def _ztail():
            pltpu.sync_copy(rbuf.at[pl.ds(0, 16)], acc.at[pl.ds(NS * WPT, 16)])

        idxs = (idx0, idx1)
        dats = (dat0, dat1)
        ls = (l0, l1)

        def issue(k, m):
            off = ebase + k * CH
            pltpu.async_copy(idx_hbm.at[pl.ds(off, CH)], idxs[m], ls[m])
            pltpu.async_copy(data_hbm.at[pl.ds(off, CH), pl.ds(dcol, C)],
                             dats[m], ls[m])

        def wait_l(m):
            pltpu.make_async_copy(idx_hbm.at[pl.ds(0, CH)],
                                  idxs[m], ls[m]).wait()
            pltpu.make_async_copy(data_hbm.at[pl.ds(0, CH), pl.ds(dcol, C)],
                                  dats[m], ls[m]).wait()

        issue(0, 0)
        plsc.subcore_barrier()

        def body(i, carry):
            for b in range(2):
                k = 2 * i + b
                wait_l(b)
                if b == 1:
                    @pl.when(i < (SCH // 2) - 1)
                    def _issue_next():
                        issue(k + 1, 0)
                else:
                    issue(k + 1, 1)
                pltpu.sync_copy(dats[b], acc.at[idxs[b]], add=True)
            return carry

        lax.fori_loop(0, SCH // 2, body, 0)
        toff = ebase + SCH * CH
        pltpu.sync_copy(idx_hbm.at[pl.ds(toff, ST)], idxt)
        pltpu.sync_copy(data_hbm.at[pl.ds(toff, ST), pl.ds(dcol, C)], datat)
        pltpu.sync_copy(datat, acc.at[idxt], add=True)
        plsc.subcore_barrier()
        for t in range(WPT // 104):
            r0 = sid * WPT + t * 104
            pltpu.sync_copy(acc.at[pl.ds(r0, 104)], rbuf)
            pltpu.sync_copy(rbuf, out_hbm.at[pl.ds(r0, 104), pl.ds(ocol, C)])

        @pl.when(sid == 0)
        def _wtail():
            pltpu.sync_copy(acc.at[pl.ds(NS * WPT, 16)], rbuf.at[pl.ds(0, 16)])
            pltpu.sync_copy(rbuf.at[pl.ds(0, 16)],
                            out_hbm.at[pl.ds(NS * WPT, 16), pl.ds(ocol, C)])

        plsc.subcore_barrier()

    ccol = cid * C
    phase(i1_hbm, ccol, pn_hbm, ccol)
    phase(i3_hbm, ccol, p3a_hbm, ccol)

    @pl.when(cid == 0)
    def _x2_phase():
        phase(i3_hbm, 2 * C, p3a_hbm, 2 * C)


BN = 1000
GN = N // BN


def _node_body(pn_ref, p3a_ref, wpp_ref, wpx_ref, wpp1_ref, bpp1_ref,
               pa_ref, dot_ref, p3t_ref):
    wpx = wpx_ref[...]
    p3a = p3a_ref[...]
    p3l0 = jnp.dot(p3a[:, 0:C], wpx, preferred_element_type=jnp.float32)
    p3l1 = jnp.dot(p3a[:, C:2 * C], wpx, preferred_element_type=jnp.float32)
    p3l2 = jnp.dot(p3a[:, 2 * C:3 * C], wpx, preferred_element_type=jnp.float32)
    dotted = p3l0 * p3l0 + p3l1 * p3l1 + p3l2 * p3l2
    p1n = jnp.tanh(
        jnp.dot(pn_ref[...], wpp_ref[...], preferred_element_type=jnp.float32))
    w1 = wpp1_ref[...]
    z = (jnp.dot(p1n, w1[0:C, :], preferred_element_type=jnp.float32)
         + jnp.dot(dotted, w1[C:2 * C, :], preferred_element_type=jnp.float32)
         + bpp1_ref[...])
    p1t1 = jnp.tanh(z)
    pa_ref[...] = p1t1[:, 0:C]
    dot_ref[...] = dotted
    pb = p1t1[:, C:2 * C]
    p3t_ref[:, 0:C] = p3l0 * pb
    p3t_ref[:, C:2 * C] = p3l1 * pb
    p3t_ref[:, 2 * C:3 * C] = p3l2 * pb


def _node_dense(p1n_raw, p3af, w_pp, w_px, w_pp1, b_pp1):
    return pl.pallas_call(
        _node_body,
        grid=(GN,),
        in_specs=[
            pl.BlockSpec((BN, 2 * C), lambda g: (g, 0)),
            pl.BlockSpec((BN, D3), lambda g: (g, 0)),
            pl.BlockSpec((2 * C, C), lambda g: (0, 0)),
            pl.BlockSpec((C, C), lambda g: (0, 0)),
            pl.BlockSpec((2 * C, 2 * C), lambda g: (0, 0)),
            pl.BlockSpec((1, 2 * C), lambda g: (0, 0)),
        ],
        out_specs=[
            pl.BlockSpec((BN, C), lambda g: (g, 0)),
            pl.BlockSpec((BN, C), lambda g: (g, 0)),
            pl.BlockSpec((BN, D3), lambda g: (g, 0)),
        ],
        out_shape=[
            jax.ShapeDtypeStruct((N, C), jnp.float32),
            jax.ShapeDtypeStruct((N, C), jnp.float32),
            jax.ShapeDtypeStruct((N, D3), jnp.float32),
        ],
    )(p1n_raw, p3af, w_pp, w_px, w_pp1, b_pp1)


def kernel(ind_2, p1, p3, d3, basis, W_pi, b_pi, W_ii, W_pp, W_pp1, b_pp1, W_px):
    ind_i = ind_2[:, 0]
    ind_j = ind_2[:, 1]
    p3f = p3.reshape(N, D3)

    w_pi_p = W_pi.reshape(2 * C, C, NB).transpose(0, 2, 1).reshape(2 * C, C * NB)
    b_pi_p = b_pi.reshape(C, NB).transpose(1, 0).reshape(1, C * NB)

    gi, gj = _gather_p1(p1, ind_i, ind_j)
    i1 = _edge_dense(gi, gj, basis, w_pi_p, b_pi_p, W_ii)
    i3f = _edge3(p3f, ind_j, d3.reshape(-1), i1)
    p1n_raw, p3af = _scatter_all(i1, i3f, ind_i)

    pxt1_a, dotted, p3tf = _node_dense(p1n_raw, p3af, W_pp, W_px, W_pp1,
                                       b_pp1.reshape(1, 2 * C))

    i3 = i3f.reshape(E, 3, C)
    p3t1 = p3tf.reshape(N, 3, C)
    return (pxt1_a, i1, i3, dotted, p3t1)
